# Initial kernel scaffold; baseline (speedup 1.0000x reference)
import jax
import jax.numpy as jnp
from jax.experimental import pallas as pl


def kernel(node_type, num_inverted_predecessors, edge_index, W_enc, b_enc, W0, b0, W1, b1, W2, b2, g0, bt0, g1, bt1, g2, bt2):
    raise NotImplementedError("write your pallas kernel here")


# speedup vs baseline: 4.5825x; 4.5825x over previous
import functools

import jax
import jax.numpy as jnp
from jax import lax
from jax.experimental import pallas as pl
from jax.experimental.pallas import tpu as pltpu
from jax.experimental.pallas import tpu_sc as plsc

NSC = 2
NTILE = 16
CH = 128
DCH = 128


def _iota16():
    return lax.iota(jnp.int32, 16)


def _sc_deg_body(E_pad, N, rows_hbm, zeros_hbm, ones_hbm, deg_hbm,
                 acc, ib, ones_v):
    cid = lax.axis_index("c")
    sid = lax.axis_index("s")
    half = N // 2
    base = cid * half
    acc_rows = acc.shape[0]
    zch = acc_rows // NTILE

    pltpu.sync_copy(zeros_hbm.at[pl.ds(0, zch)],
                    acc.at[pl.ds(sid * zch, zch)])
    pltpu.sync_copy(ones_hbm, ones_v)
    plsc.subcore_barrier()

    per_tile = E_pad // NTILE
    nch = per_tile // DCH
    tbase = sid * per_tile
    iota = _iota16()

    @pl.loop(0, nch)
    def _(k):
        off = tbase + k * DCH
        pltpu.sync_copy(rows_hbm.at[pl.ds(off, DCH)], ib)
        for j in range(DCH // 16):
            sl = pl.ds(j * 16, 16)
            r = ib[sl]
            keep = (r >= base) & (r < base + half)
            trash = half + ((off + j * 16 + iota) & (DEG_TRASH - 1))
            ib[sl] = jnp.where(keep, r - base, trash)
        pltpu.sync_copy(ones_v, acc.at[ib], add=True)

    plsc.subcore_barrier()

    per = half // NTILE
    lastn = half - (NTILE - 1) * per

    @pl.when(sid < NTILE - 1)
    def _():
        pltpu.sync_copy(acc.at[pl.ds(sid * per, per)],
                        deg_hbm.at[pl.ds(base + sid * per, per)])

    @pl.when(sid == NTILE - 1)
    def _():
        pltpu.sync_copy(
            acc.at[pl.ds((NTILE - 1) * per, lastn)],
            deg_hbm.at[pl.ds(base + (NTILE - 1) * per, lastn)])


DEG_TRASH = 128


def _sc_deg_call(rows_p, zeros128, ones_d, N):
    E_pad = rows_p.shape[0]
    half = N // 2
    acc_rows = half + DEG_TRASH
    acc_rows = ((acc_rows + NTILE * 8 - 1) // (NTILE * 8)) * (NTILE * 8)
    mesh = plsc.VectorSubcoreMesh(core_axis_name="c", subcore_axis_name="s")
    f = pl.kernel(
        functools.partial(_sc_deg_body, E_pad, N),
        out_type=jax.ShapeDtypeStruct((N, 128), jnp.float32),
        mesh=mesh,
        scratch_types=[
            pltpu.VMEM_SHARED((acc_rows, 128), jnp.float32),
            pltpu.VMEM((DCH,), jnp.int32),
            pltpu.VMEM((DCH, 128), jnp.float32),
        ],
    )
    return f(rows_p, zeros128, ones_d)


CHE = 64
SC_TRASH = 256


def _sc_scatter_body(N, E_pad, rows_hbm, cols_hbm, z2_hbm, zeros_hbm,
                     s2_hbm, acc, cr0, cr1, gl0, gl1, sl0, sl1, gb0, gb1,
                     sem0, sem1):
    cid = lax.axis_index("c")
    sid = lax.axis_index("s")
    half = N // 2
    base_c = cid * half
    acc_rows = acc.shape[0]
    zch = acc_rows // NTILE
    nsub = 2 * half

    pltpu.sync_copy(zeros_hbm.at[pl.ds(0, zch)],
                    acc.at[pl.ds(sid * zch, zch)])
    plsc.subcore_barrier()

    per_tile = E_pad // NTILE
    nch = per_tile // CHE
    tbase = sid * per_tile
    iota = _iota16()

    def load_idx(k, cr, gl, sl):
        off = tbase + k * CHE
        pltpu.sync_copy(cols_hbm.at[pl.ds(off, CHE)], cr)
        for j in range(CHE // 16):
            c = cr[pl.ds(j * 16, 16)]
            keep = (c >= base_c) & (c < base_c + half)
            lc2 = (c - base_c) * 2
            pos = (off + j * 16) * 2 + iota
            tr_lo = nsub + (pos & (SC_TRASH - 1))
            tr_hi = nsub + ((pos + 16) & (SC_TRASH - 1))
            sl[pl.ds(j * 32, 16)] = jnp.where(keep, lc2, tr_lo)
            sl[pl.ds(j * 32 + 16, 16)] = jnp.where(keep, lc2 + 1, tr_hi)
        pltpu.sync_copy(rows_hbm.at[pl.ds(off, CHE)], cr)
        for j in range(CHE // 16):
            c = sl[pl.ds(j * 32, 16)]
            keep = c < nsub
            r = cr[pl.ds(j * 16, 16)]
            r2 = r * 2
            pos = (off + j * 16) * 2 + iota
            spread = pos & 8191
            gl[pl.ds(j * 32, 16)] = jnp.where(keep, r2, spread)
            gl[pl.ds(j * 32 + 16, 16)] = jnp.where(keep, r2 + 1, spread)

    @pl.loop(0, nch)
    def _(k):
        load_idx(k, cr0, gl0, sl0)
        pltpu.async_copy(z2_hbm.at[gl0], gb0, sem0).wait()
        pltpu.sync_copy(gb0, acc.at[sl0], add=True)

    plsc.subcore_barrier()

    per = nsub // NTILE
    per = (per // 8) * 8
    lastn = nsub - (NTILE - 1) * per

    @pl.when(sid < NTILE - 1)
    def _():
        pltpu.sync_copy(
            acc.at[pl.ds(sid * per, per)],
            s2_hbm.at[pl.ds(cid * nsub + sid * per, per)])

    @pl.when(sid == NTILE - 1)
    def _():
        pltpu.sync_copy(
            acc.at[pl.ds((NTILE - 1) * per, lastn)],
            s2_hbm.at[pl.ds(cid * nsub + (NTILE - 1) * per, lastn)])


def _sc_scatter_call(rows_p, cols_p, z, zeros128):
    E_pad = rows_p.shape[0]
    N, D = z.shape
    z2 = z.reshape(2 * N, 128)
    half = N // 2
    acc_rows = 2 * half + SC_TRASH
    acc_rows = ((acc_rows + NTILE * 8 - 1) // (NTILE * 8)) * (NTILE * 8)
    mesh = plsc.VectorSubcoreMesh(core_axis_name="c", subcore_axis_name="s")
    f = pl.kernel(
        functools.partial(_sc_scatter_body, N, E_pad),
        out_type=jax.ShapeDtypeStruct((2 * N, 128), jnp.float32),
        mesh=mesh,
        scratch_types=[
            pltpu.VMEM_SHARED((acc_rows, 128), jnp.float32),
            pltpu.VMEM((CHE,), jnp.int32),
            pltpu.VMEM((CHE,), jnp.int32),
            pltpu.VMEM((2 * CHE,), jnp.int32),
            pltpu.VMEM((2 * CHE,), jnp.int32),
            pltpu.VMEM((2 * CHE,), jnp.int32),
            pltpu.VMEM((2 * CHE,), jnp.int32),
            pltpu.VMEM((2 * CHE, 128), jnp.float32),
            pltpu.VMEM((2 * CHE, 128), jnp.float32),
            pltpu.SemaphoreType.DMA,
            pltpu.SemaphoreType.DMA,
        ],
    )
    s2 = f(rows_p, cols_p, z2, zeros128)
    return s2.reshape(N, D)


def _bf16r(x):
    return x.astype(jnp.bfloat16).astype(jnp.float32)


def _tc_pre_body(nt, nip, dp, we0, we1, be, w0, b0, z0, dinv):
    d = lax.rsqrt(dp[...][:, 0:1] + 2.0)
    dinv[...] = d
    h = (_bf16r(nt[...]) * _bf16r(we0[...])
         + _bf16r(nip[...]) * _bf16r(we1[...])) + be[...]
    y = jnp.dot(h, w0[...], preferred_element_type=jnp.float32) + b0[...]
    z0[...] = d * y


def _tc_pre_call(nt, nip, deg, we0, we1, be, W0, b0, R):
    N = nt.shape[0]
    D = W0.shape[1]
    nb = N // R
    full = lambda i: (0, 0)
    return pl.pallas_call(
        _tc_pre_body,
        grid=(nb,),
        in_specs=[
            pl.BlockSpec((R, 1), lambda i: (i, 0)),
            pl.BlockSpec((R, 1), lambda i: (i, 0)),
            pl.BlockSpec((R, 128), lambda i: (i, 0)),
            pl.BlockSpec((1, D), full),
            pl.BlockSpec((1, D), full),
            pl.BlockSpec((1, D), full),
            pl.BlockSpec((D, D), full),
            pl.BlockSpec((1, D), full),
        ],
        out_specs=[
            pl.BlockSpec((R, D), lambda i: (i, 0)),
            pl.BlockSpec((R, 1), lambda i: (i, 0)),
        ],
        out_shape=[
            jax.ShapeDtypeStruct((N, D), jnp.float32),
            jax.ShapeDtypeStruct((N, 1), jnp.float32),
        ],
        compiler_params=pltpu.CompilerParams(
            dimension_semantics=("arbitrary",)),
    )(nt, nip, deg, we0, we1, be, W0, b0)


def _tc_comb_body(N, eps, s, z, dinv, g, bt, w, b, zo, ssum, ssq):
    p = pl.program_id(0)
    i = pl.program_id(1)
    d = dinv[...]
    out = d * (s[...] + z[...])

    @pl.when(p == 0)
    def _():
        @pl.when(i == 0)
        def _():
            ssum[...] = jnp.zeros_like(ssum)
            ssq[...] = jnp.zeros_like(ssq)

        ssum[...] += jnp.sum(out, axis=0, keepdims=True)
        ssq[...] += jnp.sum(out * out, axis=0, keepdims=True)

    @pl.when(p == 1)
    def _():
        mu = ssum[...] / N
        var = ssq[...] / N - mu * mu
        xn = g[...] * (out - mu) * lax.rsqrt(var + eps) + bt[...]
        h = jnp.maximum(xn, 0.0)
        zo[...] = d * (jnp.dot(h, w[...],
                               preferred_element_type=jnp.float32) + b[...])


def _tc_comb_call(s, z, dinv, g, bt, W, b, R):
    N, D = s.shape
    nb = N // R
    full = lambda p, i: (0, 0)
    return pl.pallas_call(
        functools.partial(_tc_comb_body, N, 1e-5),
        grid=(2, nb),
        in_specs=[
            pl.BlockSpec((R, D), lambda p, i: (i, 0)),
            pl.BlockSpec((R, D), lambda p, i: (i, 0)),
            pl.BlockSpec((R, 1), lambda p, i: (i, 0)),
            pl.BlockSpec((1, D), full),
            pl.BlockSpec((1, D), full),
            pl.BlockSpec((D, D), full),
            pl.BlockSpec((1, D), full),
        ],
        out_specs=pl.BlockSpec((R, D), lambda p, i: (i, 0)),
        out_shape=jax.ShapeDtypeStruct((N, D), jnp.float32),
        scratch_shapes=[
            pltpu.VMEM((1, D), jnp.float32),
            pltpu.VMEM((1, D), jnp.float32),
        ],
        compiler_params=pltpu.CompilerParams(
            dimension_semantics=("arbitrary", "arbitrary")),
    )(s, z, dinv, g, bt, W, b)


def _tc_fin_body(N, eps, s, z, dinv, g, bt, ho, ssum, ssq):
    p = pl.program_id(0)
    i = pl.program_id(1)
    out = dinv[...] * (s[...] + z[...])

    @pl.when(p == 0)
    def _():
        @pl.when(i == 0)
        def _():
            ssum[...] = jnp.zeros_like(ssum)
            ssq[...] = jnp.zeros_like(ssq)

        ssum[...] += jnp.sum(out, axis=0, keepdims=True)
        ssq[...] += jnp.sum(out * out, axis=0, keepdims=True)

    @pl.when(p == 1)
    def _():
        mu = ssum[...] / N
        var = ssq[...] / N - mu * mu
        ho[...] = g[...] * (out - mu) * lax.rsqrt(var + eps) + bt[...]


def _tc_fin_call(s, z, dinv, g, bt, R):
    N, D = s.shape
    nb = N // R
    full = lambda p, i: (0, 0)
    return pl.pallas_call(
        functools.partial(_tc_fin_body, N, 1e-5),
        grid=(2, nb),
        in_specs=[
            pl.BlockSpec((R, D), lambda p, i: (i, 0)),
            pl.BlockSpec((R, D), lambda p, i: (i, 0)),
            pl.BlockSpec((R, 1), lambda p, i: (i, 0)),
            pl.BlockSpec((1, D), full),
            pl.BlockSpec((1, D), full),
        ],
        out_specs=pl.BlockSpec((R, D), lambda p, i: (i, 0)),
        out_shape=jax.ShapeDtypeStruct((N, D), jnp.float32),
        scratch_shapes=[
            pltpu.VMEM((1, D), jnp.float32),
            pltpu.VMEM((1, D), jnp.float32),
        ],
        compiler_params=pltpu.CompilerParams(
            dimension_semantics=("arbitrary", "arbitrary")),
    )(s, z, dinv, g, bt)


def kernel(node_type, num_inverted_predecessors, edge_index, W_enc, b_enc,
           W0, b0, W1, b1, W2, b2, g0, bt0, g1, bt1, g2, bt2):
    N = node_type.shape[0]
    E = edge_index.shape[1]
    D = W0.shape[1]
    R = 2000

    rows = edge_index[0]
    cols = edge_index[1]

    unit = 2 * NTILE * CHE
    E_pad = ((E + unit - 1) // unit) * unit
    pad = E_pad - E
    rows_p = jnp.concatenate([rows, jnp.full((pad,), -1, jnp.int32)])
    cols_p = jnp.concatenate([cols, jnp.full((pad,), -1, jnp.int32)])

    half = N // 2
    acc_rows = 2 * half + SC_TRASH
    acc_rows = ((acc_rows + NTILE * 8 - 1) // (NTILE * 8)) * (NTILE * 8)
    zeros128 = jnp.zeros((acc_rows // NTILE, 128), jnp.float32)
    ones_d = jnp.ones((DCH, 128), jnp.float32)

    deg = _sc_deg_call(rows_p, zeros128, ones_d, N)

    nt = node_type.reshape(N, 1)
    nv = num_inverted_predecessors.reshape(N, 1)
    z0, dinv = _tc_pre_call(nt, nv, deg, W_enc[0:1], W_enc[1:2],
                            b_enc.reshape(1, D), W0, b0.reshape(1, D), R)
    s0 = _sc_scatter_call(rows_p, cols_p, z0, zeros128)
    z1 = _tc_comb_call(s0, z0, dinv, g0.reshape(1, D), bt0.reshape(1, D),
                       W1, b1.reshape(1, D), R)
    s1 = _sc_scatter_call(rows_p, cols_p, z1, zeros128)
    z2 = _tc_comb_call(s1, z1, dinv, g1.reshape(1, D), bt1.reshape(1, D),
                       W2, b2.reshape(1, D), R)
    s2 = _sc_scatter_call(rows_p, cols_p, z2, zeros128)
    h = _tc_fin_call(s2, z2, dinv, g2.reshape(1, D), bt2.reshape(1, D), R)
    return h


# speedup vs baseline: 4.7865x; 1.0445x over previous
import functools

import jax
import jax.numpy as jnp
from jax import lax
from jax.experimental import pallas as pl
from jax.experimental.pallas import tpu as pltpu
from jax.experimental.pallas import tpu_sc as plsc

NSC = 2
NTILE = 16
CH = 128
DCH = 128


def _iota16():
    return lax.iota(jnp.int32, 16)


def _sc_deg_body(E_pad, N, rows_hbm, zeros_hbm, ones_hbm, deg_hbm,
                 acc, ib, ones_v):
    cid = lax.axis_index("c")
    sid = lax.axis_index("s")
    half = N // 2
    base = cid * half
    acc_rows = acc.shape[0]
    zch = acc_rows // NTILE

    pltpu.sync_copy(zeros_hbm.at[pl.ds(0, zch)],
                    acc.at[pl.ds(sid * zch, zch)])
    pltpu.sync_copy(ones_hbm, ones_v)
    plsc.subcore_barrier()

    per_tile = E_pad // NTILE
    nch = per_tile // DCH
    tbase = sid * per_tile

    @pl.loop(0, nch)
    def _(k):
        off = tbase + k * DCH
        pltpu.sync_copy(rows_hbm.at[pl.ds(off, DCH)], ib)
        for j in range(DCH // 16):
            sl = pl.ds(j * 16, 16)
            r = ib[sl]
            keep = (r >= base) & (r < base + half)
            ib[sl] = jnp.where(keep, r - base, -1)
        pltpu.sync_copy(
            ones_v, acc.at[plsc.Indices(ib, ignored_value=-1)], add=True)

    plsc.subcore_barrier()

    per = half // NTILE
    lastn = half - (NTILE - 1) * per

    @pl.when(sid < NTILE - 1)
    def _():
        pltpu.sync_copy(acc.at[pl.ds(sid * per, per)],
                        deg_hbm.at[pl.ds(base + sid * per, per)])

    @pl.when(sid == NTILE - 1)
    def _():
        pltpu.sync_copy(
            acc.at[pl.ds((NTILE - 1) * per, lastn)],
            deg_hbm.at[pl.ds(base + (NTILE - 1) * per, lastn)])


DEG_TRASH = 0


def _sc_deg_call(rows_p, zeros128, ones_d, N):
    E_pad = rows_p.shape[0]
    half = N // 2
    acc_rows = ((half + NTILE * 8 - 1) // (NTILE * 8)) * (NTILE * 8)
    mesh = plsc.VectorSubcoreMesh(core_axis_name="c", subcore_axis_name="s")
    f = pl.kernel(
        functools.partial(_sc_deg_body, E_pad, N),
        out_type=jax.ShapeDtypeStruct((N, 128), jnp.float32),
        mesh=mesh,
        scratch_types=[
            pltpu.VMEM_SHARED((acc_rows, 128), jnp.float32),
            pltpu.VMEM((DCH,), jnp.int32),
            pltpu.VMEM((DCH, 128), jnp.float32),
        ],
    )
    return f(rows_p, zeros128, ones_d)


CHE = 64
SC_TRASH = 0


def _sc_scatter_body(N, E_pad, rows_hbm, cols_hbm, z2_hbm, zeros_hbm,
                     s2_hbm, acc, cr0, cr1, gl0, gl1, sl0, sl1, gb0, gb1,
                     sem0, sem1):
    cid = lax.axis_index("c")
    sid = lax.axis_index("s")
    half = N // 2
    base_c = cid * half
    acc_rows = acc.shape[0]
    zch = acc_rows // NTILE
    nsub = 2 * half

    pltpu.sync_copy(zeros_hbm.at[pl.ds(0, zch)],
                    acc.at[pl.ds(sid * zch, zch)])
    plsc.subcore_barrier()

    per_tile = E_pad // NTILE
    nch = per_tile // CHE
    tbase = sid * per_tile

    def load_idx(k, cr, gl, sl):
        off = tbase + k * CHE
        pltpu.sync_copy(cols_hbm.at[pl.ds(off, CHE)], cr)
        for j in range(CHE // 16):
            c = cr[pl.ds(j * 16, 16)]
            keep = (c >= base_c) & (c < base_c + half)
            lc2 = (c - base_c) * 2
            sl[pl.ds(j * 32, 16)] = jnp.where(keep, lc2, -1)
            sl[pl.ds(j * 32 + 16, 16)] = jnp.where(keep, lc2 + 1, -1)
        pltpu.sync_copy(rows_hbm.at[pl.ds(off, CHE)], cr)
        for j in range(CHE // 16):
            keep = sl[pl.ds(j * 32, 16)] >= 0
            r = cr[pl.ds(j * 16, 16)]
            r2 = r * 2
            gl[pl.ds(j * 32, 16)] = jnp.where(keep, r2, -1)
            gl[pl.ds(j * 32 + 16, 16)] = jnp.where(keep, r2 + 1, -1)

    @pl.loop(0, nch)
    def _(k):
        load_idx(k, cr0, gl0, sl0)
        pltpu.async_copy(
            z2_hbm.at[plsc.Indices(gl0, ignored_value=-1)], gb0, sem0).wait()
        pltpu.sync_copy(
            gb0, acc.at[plsc.Indices(sl0, ignored_value=-1)], add=True)

    plsc.subcore_barrier()

    per = nsub // NTILE
    per = (per // 8) * 8
    lastn = nsub - (NTILE - 1) * per

    @pl.when(sid < NTILE - 1)
    def _():
        pltpu.sync_copy(
            acc.at[pl.ds(sid * per, per)],
            s2_hbm.at[pl.ds(cid * nsub + sid * per, per)])

    @pl.when(sid == NTILE - 1)
    def _():
        pltpu.sync_copy(
            acc.at[pl.ds((NTILE - 1) * per, lastn)],
            s2_hbm.at[pl.ds(cid * nsub + (NTILE - 1) * per, lastn)])


def _sc_scatter_call(rows_p, cols_p, z, zeros128):
    E_pad = rows_p.shape[0]
    N, D = z.shape
    z2 = z.reshape(2 * N, 128)
    half = N // 2
    acc_rows = ((2 * half + NTILE * 8 - 1) // (NTILE * 8)) * (NTILE * 8)
    mesh = plsc.VectorSubcoreMesh(core_axis_name="c", subcore_axis_name="s")
    f = pl.kernel(
        functools.partial(_sc_scatter_body, N, E_pad),
        out_type=jax.ShapeDtypeStruct((2 * N, 128), jnp.float32),
        mesh=mesh,
        scratch_types=[
            pltpu.VMEM_SHARED((acc_rows, 128), jnp.float32),
            pltpu.VMEM((CHE,), jnp.int32),
            pltpu.VMEM((CHE,), jnp.int32),
            pltpu.VMEM((2 * CHE,), jnp.int32),
            pltpu.VMEM((2 * CHE,), jnp.int32),
            pltpu.VMEM((2 * CHE,), jnp.int32),
            pltpu.VMEM((2 * CHE,), jnp.int32),
            pltpu.VMEM((2 * CHE, 128), jnp.float32),
            pltpu.VMEM((2 * CHE, 128), jnp.float32),
            pltpu.SemaphoreType.DMA,
            pltpu.SemaphoreType.DMA,
        ],
    )
    s2 = f(rows_p, cols_p, z2, zeros128)
    return s2.reshape(N, D)


def _bf16r(x):
    return x.astype(jnp.bfloat16).astype(jnp.float32)


def _tc_pre_body(nt, nip, dp, we0, we1, be, w0, b0, z0, dinv):
    d = lax.rsqrt(dp[...][:, 0:1] + 2.0)
    dinv[...] = d
    h = (_bf16r(nt[...]) * _bf16r(we0[...])
         + _bf16r(nip[...]) * _bf16r(we1[...])) + be[...]
    y = jnp.dot(h, w0[...], preferred_element_type=jnp.float32) + b0[...]
    z0[...] = d * y


def _tc_pre_call(nt, nip, deg, we0, we1, be, W0, b0, R):
    N = nt.shape[0]
    D = W0.shape[1]
    nb = N // R
    full = lambda i: (0, 0)
    return pl.pallas_call(
        _tc_pre_body,
        grid=(nb,),
        in_specs=[
            pl.BlockSpec((R, 1), lambda i: (i, 0)),
            pl.BlockSpec((R, 1), lambda i: (i, 0)),
            pl.BlockSpec((R, 128), lambda i: (i, 0)),
            pl.BlockSpec((1, D), full),
            pl.BlockSpec((1, D), full),
            pl.BlockSpec((1, D), full),
            pl.BlockSpec((D, D), full),
            pl.BlockSpec((1, D), full),
        ],
        out_specs=[
            pl.BlockSpec((R, D), lambda i: (i, 0)),
            pl.BlockSpec((R, 1), lambda i: (i, 0)),
        ],
        out_shape=[
            jax.ShapeDtypeStruct((N, D), jnp.float32),
            jax.ShapeDtypeStruct((N, 1), jnp.float32),
        ],
        compiler_params=pltpu.CompilerParams(
            dimension_semantics=("arbitrary",)),
    )(nt, nip, deg, we0, we1, be, W0, b0)


def _tc_comb_body(N, eps, s, z, dinv, g, bt, w, b, zo, ssum, ssq):
    p = pl.program_id(0)
    i = pl.program_id(1)
    d = dinv[...]
    out = d * (s[...] + z[...])

    @pl.when(p == 0)
    def _():
        @pl.when(i == 0)
        def _():
            ssum[...] = jnp.zeros_like(ssum)
            ssq[...] = jnp.zeros_like(ssq)

        ssum[...] += jnp.sum(out, axis=0, keepdims=True)
        ssq[...] += jnp.sum(out * out, axis=0, keepdims=True)

    @pl.when(p == 1)
    def _():
        mu = ssum[...] / N
        var = ssq[...] / N - mu * mu
        xn = g[...] * (out - mu) * lax.rsqrt(var + eps) + bt[...]
        h = jnp.maximum(xn, 0.0)
        zo[...] = d * (jnp.dot(h, w[...],
                               preferred_element_type=jnp.float32) + b[...])


def _tc_comb_call(s, z, dinv, g, bt, W, b, R):
    N, D = s.shape
    nb = N // R
    full = lambda p, i: (0, 0)
    return pl.pallas_call(
        functools.partial(_tc_comb_body, N, 1e-5),
        grid=(2, nb),
        in_specs=[
            pl.BlockSpec((R, D), lambda p, i: (i, 0)),
            pl.BlockSpec((R, D), lambda p, i: (i, 0)),
            pl.BlockSpec((R, 1), lambda p, i: (i, 0)),
            pl.BlockSpec((1, D), full),
            pl.BlockSpec((1, D), full),
            pl.BlockSpec((D, D), full),
            pl.BlockSpec((1, D), full),
        ],
        out_specs=pl.BlockSpec((R, D), lambda p, i: (i, 0)),
        out_shape=jax.ShapeDtypeStruct((N, D), jnp.float32),
        scratch_shapes=[
            pltpu.VMEM((1, D), jnp.float32),
            pltpu.VMEM((1, D), jnp.float32),
        ],
        compiler_params=pltpu.CompilerParams(
            dimension_semantics=("arbitrary", "arbitrary")),
    )(s, z, dinv, g, bt, W, b)


def _tc_fin_body(N, eps, s, z, dinv, g, bt, ho, ssum, ssq):
    p = pl.program_id(0)
    i = pl.program_id(1)
    out = dinv[...] * (s[...] + z[...])

    @pl.when(p == 0)
    def _():
        @pl.when(i == 0)
        def _():
            ssum[...] = jnp.zeros_like(ssum)
            ssq[...] = jnp.zeros_like(ssq)

        ssum[...] += jnp.sum(out, axis=0, keepdims=True)
        ssq[...] += jnp.sum(out * out, axis=0, keepdims=True)

    @pl.when(p == 1)
    def _():
        mu = ssum[...] / N
        var = ssq[...] / N - mu * mu
        ho[...] = g[...] * (out - mu) * lax.rsqrt(var + eps) + bt[...]


def _tc_fin_call(s, z, dinv, g, bt, R):
    N, D = s.shape
    nb = N // R
    full = lambda p, i: (0, 0)
    return pl.pallas_call(
        functools.partial(_tc_fin_body, N, 1e-5),
        grid=(2, nb),
        in_specs=[
            pl.BlockSpec((R, D), lambda p, i: (i, 0)),
            pl.BlockSpec((R, D), lambda p, i: (i, 0)),
            pl.BlockSpec((R, 1), lambda p, i: (i, 0)),
            pl.BlockSpec((1, D), full),
            pl.BlockSpec((1, D), full),
        ],
        out_specs=pl.BlockSpec((R, D), lambda p, i: (i, 0)),
        out_shape=jax.ShapeDtypeStruct((N, D), jnp.float32),
        scratch_shapes=[
            pltpu.VMEM((1, D), jnp.float32),
            pltpu.VMEM((1, D), jnp.float32),
        ],
        compiler_params=pltpu.CompilerParams(
            dimension_semantics=("arbitrary", "arbitrary")),
    )(s, z, dinv, g, bt)


def kernel(node_type, num_inverted_predecessors, edge_index, W_enc, b_enc,
           W0, b0, W1, b1, W2, b2, g0, bt0, g1, bt1, g2, bt2):
    N = node_type.shape[0]
    E = edge_index.shape[1]
    D = W0.shape[1]
    R = 2000

    rows = edge_index[0]
    cols = edge_index[1]

    unit = 2 * NTILE * CHE
    E_pad = ((E + unit - 1) // unit) * unit
    pad = E_pad - E
    rows_p = jnp.concatenate([rows, jnp.full((pad,), -1, jnp.int32)])
    cols_p = jnp.concatenate([cols, jnp.full((pad,), -1, jnp.int32)])

    half = N // 2
    acc_rows = ((2 * half + NTILE * 8 - 1) // (NTILE * 8)) * (NTILE * 8)
    zeros128 = jnp.zeros((acc_rows // NTILE, 128), jnp.float32)
    ones_d = jnp.ones((DCH, 128), jnp.float32)

    deg = _sc_deg_call(rows_p, zeros128, ones_d, N)

    nt = node_type.reshape(N, 1)
    nv = num_inverted_predecessors.reshape(N, 1)
    z0, dinv = _tc_pre_call(nt, nv, deg, W_enc[0:1], W_enc[1:2],
                            b_enc.reshape(1, D), W0, b0.reshape(1, D), R)
    s0 = _sc_scatter_call(rows_p, cols_p, z0, zeros128)
    z1 = _tc_comb_call(s0, z0, dinv, g0.reshape(1, D), bt0.reshape(1, D),
                       W1, b1.reshape(1, D), R)
    s1 = _sc_scatter_call(rows_p, cols_p, z1, zeros128)
    z2 = _tc_comb_call(s1, z1, dinv, g1.reshape(1, D), bt1.reshape(1, D),
                       W2, b2.reshape(1, D), R)
    s2 = _sc_scatter_call(rows_p, cols_p, z2, zeros128)
    h = _tc_fin_call(s2, z2, dinv, g2.reshape(1, D), bt2.reshape(1, D), R)
    return h


# speedup vs baseline: 7.3102x; 1.5273x over previous
import functools

import jax
import jax.numpy as jnp
from jax import lax
from jax.experimental import pallas as pl
from jax.experimental.pallas import tpu as pltpu
from jax.experimental.pallas import tpu_sc as plsc

NSC = 2
NTILE = 16
CH = 128
DCH = 128


def _iota16():
    return lax.iota(jnp.int32, 16)


def _sc_deg_body(E_pad, N, rows_hbm, zeros_hbm, ones_hbm, deg_hbm,
                 acc, ib, ones_v):
    cid = lax.axis_index("c")
    sid = lax.axis_index("s")
    half = N // 2
    base = cid * half
    acc_rows = acc.shape[0]
    zch = acc_rows // NTILE

    pltpu.sync_copy(zeros_hbm.at[pl.ds(0, zch)],
                    acc.at[pl.ds(sid * zch, zch)])
    pltpu.sync_copy(ones_hbm, ones_v)
    plsc.subcore_barrier()

    per_tile = E_pad // NTILE
    nch = per_tile // DCH
    tbase = sid * per_tile

    @pl.loop(0, nch)
    def _(k):
        off = tbase + k * DCH
        pltpu.sync_copy(rows_hbm.at[pl.ds(off, DCH)], ib)
        for j in range(DCH // 16):
            sl = pl.ds(j * 16, 16)
            r = ib[sl]
            keep = (r >= base) & (r < base + half)
            ib[sl] = jnp.where(keep, r - base, -1)
        pltpu.sync_copy(
            ones_v, acc.at[plsc.Indices(ib, ignored_value=-1)], add=True)

    plsc.subcore_barrier()

    per = half // NTILE
    lastn = half - (NTILE - 1) * per

    @pl.when(sid < NTILE - 1)
    def _():
        pltpu.sync_copy(acc.at[pl.ds(sid * per, per)],
                        deg_hbm.at[pl.ds(base + sid * per, per)])

    @pl.when(sid == NTILE - 1)
    def _():
        pltpu.sync_copy(
            acc.at[pl.ds((NTILE - 1) * per, lastn)],
            deg_hbm.at[pl.ds(base + (NTILE - 1) * per, lastn)])


DEG_TRASH = 0


def _sc_deg_call(rows_p, zeros128, ones_d, N):
    E_pad = rows_p.shape[0]
    half = N // 2
    acc_rows = ((half + NTILE * 8 - 1) // (NTILE * 8)) * (NTILE * 8)
    mesh = plsc.VectorSubcoreMesh(core_axis_name="c", subcore_axis_name="s")
    f = pl.kernel(
        functools.partial(_sc_deg_body, E_pad, N),
        out_type=jax.ShapeDtypeStruct((N, 128), jnp.float32),
        mesh=mesh,
        scratch_types=[
            pltpu.VMEM_SHARED((acc_rows, 128), jnp.float32),
            pltpu.VMEM((DCH,), jnp.int32),
            pltpu.VMEM((DCH, 128), jnp.float32),
        ],
    )
    return f(rows_p, zeros128, ones_d)


CHE = 64
SC_TRASH = 0


def _sc_scatter_body(N, E_pad, rows_hbm, cols_hbm, z2_hbm, zeros_hbm,
                     s2_hbm, acc, cr0, cr1, gl0, gl1, sl0, sl1, gb0, gb1,
                     sem0, sem1):
    cid = lax.axis_index("c")
    sid = lax.axis_index("s")
    half = N // 2
    base_c = cid * half
    acc_rows = acc.shape[0]
    zch = acc_rows // NTILE
    nsub = 2 * half

    pltpu.sync_copy(zeros_hbm.at[pl.ds(0, zch)],
                    acc.at[pl.ds(sid * zch, zch)])
    plsc.subcore_barrier()

    per_tile = E_pad // NTILE
    nch = per_tile // CHE
    tbase = sid * per_tile

    def load_idx(k, cr, gl, sl):
        off = tbase + k * CHE
        pltpu.sync_copy(cols_hbm.at[pl.ds(off, CHE)], cr)
        for j in range(CHE // 16):
            c = cr[pl.ds(j * 16, 16)]
            keep = (c >= base_c) & (c < base_c + half)
            lc2 = (c - base_c) * 2
            sl[pl.ds(j * 32, 16)] = jnp.where(keep, lc2, -1)
            sl[pl.ds(j * 32 + 16, 16)] = jnp.where(keep, lc2 + 1, -1)
        pltpu.sync_copy(rows_hbm.at[pl.ds(off, CHE)], cr)
        for j in range(CHE // 16):
            keep = sl[pl.ds(j * 32, 16)] >= 0
            r = cr[pl.ds(j * 16, 16)]
            r2 = r * 2
            gl[pl.ds(j * 32, 16)] = jnp.where(keep, r2, -1)
            gl[pl.ds(j * 32 + 16, 16)] = jnp.where(keep, r2 + 1, -1)

    def fire(gl, gb, sem):
        pltpu.async_copy(
            z2_hbm.at[plsc.Indices(gl, ignored_value=-1)], gb, sem)

    def drain(gl, gb, sem):
        pltpu.make_async_copy(
            z2_hbm.at[plsc.Indices(gl, ignored_value=-1)], gb, sem).wait()

    bufs = ((cr0, gl0, sl0, gb0, sem0), (cr1, gl1, sl1, gb1, sem1))

    load_idx(0, cr0, gl0, sl0)
    fire(gl0, gb0, sem0)

    @pl.loop(0, nch, step=2)
    def _(k2):
        for b in range(2):
            k = k2 + b
            cr_c, gl_c, sl_c, gb_c, sem_c = bufs[b]
            cr_n, gl_n, sl_n, gb_n, sem_n = bufs[1 - b]
            kn = jnp.minimum(k + 1, nch - 1)
            load_idx(kn, cr_n, gl_n, sl_n)
            fire(gl_n, gb_n, sem_n)
            drain(gl_c, gb_c, sem_c)
            pltpu.sync_copy(
                gb_c, acc.at[plsc.Indices(sl_c, ignored_value=-1)], add=True)

    drain(gl0, gb0, sem0)

    plsc.subcore_barrier()

    per = nsub // NTILE
    per = (per // 8) * 8
    lastn = nsub - (NTILE - 1) * per

    @pl.when(sid < NTILE - 1)
    def _():
        pltpu.sync_copy(
            acc.at[pl.ds(sid * per, per)],
            s2_hbm.at[pl.ds(cid * nsub + sid * per, per)])

    @pl.when(sid == NTILE - 1)
    def _():
        pltpu.sync_copy(
            acc.at[pl.ds((NTILE - 1) * per, lastn)],
            s2_hbm.at[pl.ds(cid * nsub + (NTILE - 1) * per, lastn)])


def _sc_scatter_call(rows_p, cols_p, z, zeros128):
    E_pad = rows_p.shape[0]
    N, D = z.shape
    z2 = z.reshape(2 * N, 128)
    half = N // 2
    acc_rows = ((2 * half + NTILE * 8 - 1) // (NTILE * 8)) * (NTILE * 8)
    mesh = plsc.VectorSubcoreMesh(core_axis_name="c", subcore_axis_name="s")
    f = pl.kernel(
        functools.partial(_sc_scatter_body, N, E_pad),
        out_type=jax.ShapeDtypeStruct((2 * N, 128), jnp.float32),
        mesh=mesh,
        scratch_types=[
            pltpu.VMEM_SHARED((acc_rows, 128), jnp.float32),
            pltpu.VMEM((CHE,), jnp.int32),
            pltpu.VMEM((CHE,), jnp.int32),
            pltpu.VMEM((2 * CHE,), jnp.int32),
            pltpu.VMEM((2 * CHE,), jnp.int32),
            pltpu.VMEM((2 * CHE,), jnp.int32),
            pltpu.VMEM((2 * CHE,), jnp.int32),
            pltpu.VMEM((2 * CHE, 128), jnp.float32),
            pltpu.VMEM((2 * CHE, 128), jnp.float32),
            pltpu.SemaphoreType.DMA,
            pltpu.SemaphoreType.DMA,
        ],
    )
    s2 = f(rows_p, cols_p, z2, zeros128)
    return s2.reshape(N, D)


def _bf16r(x):
    return x.astype(jnp.bfloat16).astype(jnp.float32)


def _tc_pre_body(nt, nip, dp, we0, we1, be, w0, b0, z0, dinv):
    d = lax.rsqrt(dp[...][:, 0:1] + 2.0)
    dinv[...] = d
    h = (_bf16r(nt[...]) * _bf16r(we0[...])
         + _bf16r(nip[...]) * _bf16r(we1[...])) + be[...]
    y = jnp.dot(h, w0[...], preferred_element_type=jnp.float32) + b0[...]
    z0[...] = d * y


def _tc_pre_call(nt, nip, deg, we0, we1, be, W0, b0, R):
    N = nt.shape[0]
    D = W0.shape[1]
    nb = N // R
    full = lambda i: (0, 0)
    return pl.pallas_call(
        _tc_pre_body,
        grid=(nb,),
        in_specs=[
            pl.BlockSpec((R, 1), lambda i: (i, 0)),
            pl.BlockSpec((R, 1), lambda i: (i, 0)),
            pl.BlockSpec((R, 128), lambda i: (i, 0)),
            pl.BlockSpec((1, D), full),
            pl.BlockSpec((1, D), full),
            pl.BlockSpec((1, D), full),
            pl.BlockSpec((D, D), full),
            pl.BlockSpec((1, D), full),
        ],
        out_specs=[
            pl.BlockSpec((R, D), lambda i: (i, 0)),
            pl.BlockSpec((R, 1), lambda i: (i, 0)),
        ],
        out_shape=[
            jax.ShapeDtypeStruct((N, D), jnp.float32),
            jax.ShapeDtypeStruct((N, 1), jnp.float32),
        ],
        compiler_params=pltpu.CompilerParams(
            dimension_semantics=("arbitrary",)),
    )(nt, nip, deg, we0, we1, be, W0, b0)


def _tc_comb_body(N, eps, s, z, dinv, g, bt, w, b, zo, ssum, ssq):
    p = pl.program_id(0)
    i = pl.program_id(1)
    d = dinv[...]
    out = d * (s[...] + z[...])

    @pl.when(p == 0)
    def _():
        @pl.when(i == 0)
        def _():
            ssum[...] = jnp.zeros_like(ssum)
            ssq[...] = jnp.zeros_like(ssq)

        ssum[...] += jnp.sum(out, axis=0, keepdims=True)
        ssq[...] += jnp.sum(out * out, axis=0, keepdims=True)

    @pl.when(p == 1)
    def _():
        mu = ssum[...] / N
        var = ssq[...] / N - mu * mu
        xn = g[...] * (out - mu) * lax.rsqrt(var + eps) + bt[...]
        h = jnp.maximum(xn, 0.0)
        zo[...] = d * (jnp.dot(h, w[...],
                               preferred_element_type=jnp.float32) + b[...])


def _tc_comb_call(s, z, dinv, g, bt, W, b, R):
    N, D = s.shape
    nb = N // R
    full = lambda p, i: (0, 0)
    return pl.pallas_call(
        functools.partial(_tc_comb_body, N, 1e-5),
        grid=(2, nb),
        in_specs=[
            pl.BlockSpec((R, D), lambda p, i: (i, 0)),
            pl.BlockSpec((R, D), lambda p, i: (i, 0)),
            pl.BlockSpec((R, 1), lambda p, i: (i, 0)),
            pl.BlockSpec((1, D), full),
            pl.BlockSpec((1, D), full),
            pl.BlockSpec((D, D), full),
            pl.BlockSpec((1, D), full),
        ],
        out_specs=pl.BlockSpec((R, D), lambda p, i: (i, 0)),
        out_shape=jax.ShapeDtypeStruct((N, D), jnp.float32),
        scratch_shapes=[
            pltpu.VMEM((1, D), jnp.float32),
            pltpu.VMEM((1, D), jnp.float32),
        ],
        compiler_params=pltpu.CompilerParams(
            dimension_semantics=("arbitrary", "arbitrary")),
    )(s, z, dinv, g, bt, W, b)


def _tc_fin_body(N, eps, s, z, dinv, g, bt, ho, ssum, ssq):
    p = pl.program_id(0)
    i = pl.program_id(1)
    out = dinv[...] * (s[...] + z[...])

    @pl.when(p == 0)
    def _():
        @pl.when(i == 0)
        def _():
            ssum[...] = jnp.zeros_like(ssum)
            ssq[...] = jnp.zeros_like(ssq)

        ssum[...] += jnp.sum(out, axis=0, keepdims=True)
        ssq[...] += jnp.sum(out * out, axis=0, keepdims=True)

    @pl.when(p == 1)
    def _():
        mu = ssum[...] / N
        var = ssq[...] / N - mu * mu
        ho[...] = g[...] * (out - mu) * lax.rsqrt(var + eps) + bt[...]


def _tc_fin_call(s, z, dinv, g, bt, R):
    N, D = s.shape
    nb = N // R
    full = lambda p, i: (0, 0)
    return pl.pallas_call(
        functools.partial(_tc_fin_body, N, 1e-5),
        grid=(2, nb),
        in_specs=[
            pl.BlockSpec((R, D), lambda p, i: (i, 0)),
            pl.BlockSpec((R, D), lambda p, i: (i, 0)),
            pl.BlockSpec((R, 1), lambda p, i: (i, 0)),
            pl.BlockSpec((1, D), full),
            pl.BlockSpec((1, D), full),
        ],
        out_specs=pl.BlockSpec((R, D), lambda p, i: (i, 0)),
        out_shape=jax.ShapeDtypeStruct((N, D), jnp.float32),
        scratch_shapes=[
            pltpu.VMEM((1, D), jnp.float32),
            pltpu.VMEM((1, D), jnp.float32),
        ],
        compiler_params=pltpu.CompilerParams(
            dimension_semantics=("arbitrary", "arbitrary")),
    )(s, z, dinv, g, bt)


def kernel(node_type, num_inverted_predecessors, edge_index, W_enc, b_enc,
           W0, b0, W1, b1, W2, b2, g0, bt0, g1, bt1, g2, bt2):
    N = node_type.shape[0]
    E = edge_index.shape[1]
    D = W0.shape[1]
    R = 2000

    rows = edge_index[0]
    cols = edge_index[1]

    unit = 2 * NTILE * CHE
    E_pad = ((E + unit - 1) // unit) * unit
    pad = E_pad - E
    rows_p = jnp.concatenate([rows, jnp.full((pad,), -1, jnp.int32)])
    cols_p = jnp.concatenate([cols, jnp.full((pad,), -1, jnp.int32)])

    half = N // 2
    acc_rows = ((2 * half + NTILE * 8 - 1) // (NTILE * 8)) * (NTILE * 8)
    zeros128 = jnp.zeros((acc_rows // NTILE, 128), jnp.float32)
    ones_d = jnp.ones((DCH, 128), jnp.float32)

    deg = _sc_deg_call(rows_p, zeros128, ones_d, N)

    nt = node_type.reshape(N, 1)
    nv = num_inverted_predecessors.reshape(N, 1)
    z0, dinv = _tc_pre_call(nt, nv, deg, W_enc[0:1], W_enc[1:2],
                            b_enc.reshape(1, D), W0, b0.reshape(1, D), R)
    s0 = _sc_scatter_call(rows_p, cols_p, z0, zeros128)
    z1 = _tc_comb_call(s0, z0, dinv, g0.reshape(1, D), bt0.reshape(1, D),
                       W1, b1.reshape(1, D), R)
    s1 = _sc_scatter_call(rows_p, cols_p, z1, zeros128)
    z2 = _tc_comb_call(s1, z1, dinv, g1.reshape(1, D), bt1.reshape(1, D),
                       W2, b2.reshape(1, D), R)
    s2 = _sc_scatter_call(rows_p, cols_p, z2, zeros128)
    h = _tc_fin_call(s2, z2, dinv, g2.reshape(1, D), bt2.reshape(1, D), R)
    return h
